# baseline (speedup 1.0000x reference)
import functools

import jax
import jax.numpy as jnp
from jax import lax
from jax.experimental import pallas as pl
from jax.experimental.pallas import tpu as pltpu
from jax.experimental.pallas import tpu_sc as plsc

ROW_TILE = 4096


def _split3(v):
    s1 = v.astype(jnp.bfloat16)
    r1 = v - s1.astype(jnp.float32)
    s2 = r1.astype(jnp.bfloat16)
    r2 = r1 - s2.astype(jnp.float32)
    s3 = r2.astype(jnp.bfloat16)
    return s1, s2, s3


def _dist_argmin_body(x_ref, cb_ref, idx_ref, loss_ref,
                      cba_ref, iota_ref):
    i = pl.program_id(0)
    x = x_ref[:]
    t = x.shape[0]
    d = x.shape[1]

    @pl.when(i == 0)
    def _prep():
        cb = cb_ref[:]
        c2 = jnp.sum(cb * cb, axis=1, keepdims=True)
        c1s, c2s, c3s = _split3(-2.0 * cb)
        q1, q2, q3 = _split3(c2)
        cba_ref[:] = jnp.concatenate(
            [c1s, c2s, c2s, c1s, c3s, c1s, q1, q2, q3], axis=1)
        iota_ref[:] = lax.broadcasted_iota(
            jnp.int32, iota_ref.shape, 1).astype(jnp.float32)

    x1, x2, x3 = _split3(x)
    ones = jnp.ones((t, 3), jnp.bfloat16)
    xa = jnp.concatenate([x1, x2, x1, x2, x1, x3, ones], axis=1)
    score = lax.dot_general(
        xa, cba_ref[:], (((1,), (1,)), ((), ())),
        preferred_element_type=jnp.float32)
    m = jnp.min(score, axis=1, keepdims=True)
    n_k = score.shape[1]
    idx = jnp.min(jnp.where(score == m, iota_ref[:], float(n_k)),
                  axis=1, keepdims=True)
    idx_ref[:] = idx.astype(jnp.int32)

    x2 = jnp.sum(x * x, axis=1, keepdims=True)
    part = jnp.sum(m + x2)

    @pl.when(i == 0)
    def _init():
        loss_ref[0] = 0.0

    loss_ref[0] = loss_ref[0] + part


def _distances_and_argmin(x, codebook):
    b, d = x.shape
    k = codebook.shape[0]
    tile = min(ROW_TILE, b)
    grid = (b // tile,)
    idx, loss_sum = pl.pallas_call(
        _dist_argmin_body,
        grid=grid,
        in_specs=[
            pl.BlockSpec((tile, d), lambda i: (i, 0)),
            pl.BlockSpec((k, d), lambda i: (0, 0)),
        ],
        out_specs=[
            pl.BlockSpec((tile, 1), lambda i: (i, 0)),
            pl.BlockSpec(memory_space=pltpu.SMEM),
        ],
        out_shape=[
            jax.ShapeDtypeStruct((b, 1), jnp.int32),
            jax.ShapeDtypeStruct((1,), jnp.float32),
        ],
        scratch_shapes=[pltpu.VMEM((k, 6 * d + 3), jnp.bfloat16),
                        pltpu.VMEM((tile, k), jnp.float32)],
    )(x, codebook)
    return idx.reshape(b), loss_sum[0]


def _sc_gather(codebook, idx):
    k, d = codebook.shape
    b = idx.shape[0]
    info = plsc.get_sparse_core_info()
    nc, ns = info.num_cores, info.num_subcores
    nw = nc * ns
    b_per_w = b // nw
    ch = 128
    n_ch = b_per_w // ch
    idx2d = idx.reshape(b // ch, ch)
    mesh = plsc.VectorSubcoreMesh(core_axis_name="c", subcore_axis_name="s")

    @functools.partial(
        pl.kernel,
        mesh=mesh,
        compiler_params=pltpu.CompilerParams(use_tc_tiling_on_sc=False),
        out_type=jax.ShapeDtypeStruct((b, d), jnp.float32),
        scratch_types=[
            pltpu.VMEM((n_ch, ch), jnp.int32),
            pltpu.VMEM((b_per_w, d), jnp.float32),
            pltpu.SemaphoreType.DMA,
            pltpu.SemaphoreType.DMA,
        ],
    )
    def gather_kernel(table_hbm, idx_hbm, out_hbm, idx_v, rows_v, sem_g, sem_o):
        wid = lax.axis_index("s") * nc + lax.axis_index("c")
        base = wid * b_per_w
        pltpu.sync_copy(idx_hbm.at[pl.ds(wid * n_ch, n_ch)], idx_v)
        gathers = []
        for j in range(n_ch):
            gathers.append(pltpu.async_copy(
                table_hbm.at[idx_v.at[j]],
                rows_v.at[pl.ds(j * ch, ch)], sem_g))
        outs = []
        for j in range(n_ch):
            gathers[j].wait()
            outs.append(pltpu.async_copy(
                rows_v.at[pl.ds(j * ch, ch)],
                out_hbm.at[pl.ds(base + j * ch, ch)], sem_o))
        for c in outs:
            c.wait()

    return gather_kernel(codebook, idx2d)


def kernel(x, codebook):
    b, d = x.shape
    h = b // 2
    idx_a, sum_a = _distances_and_argmin(x[:h], codebook)
    quant_a = _sc_gather(codebook, idx_a)
    idx_b, sum_b = _distances_and_argmin(x[h:], codebook)
    quant_b = _sc_gather(codebook, idx_b)
    loss = (sum_a + sum_b) / (b * d)
    quant = jnp.concatenate([quant_a, quant_b], axis=0)
    idx = jnp.concatenate([idx_a, idx_b], axis=0)
    return quant, loss, loss, idx


# speedup vs baseline: 1.3239x; 1.3239x over previous
import functools

import jax
import jax.numpy as jnp
from jax import lax
from jax.experimental import pallas as pl
from jax.experimental.pallas import tpu as pltpu
from jax.experimental.pallas import tpu_sc as plsc

ROW_TILE = 4096


def _split3(v):
    s1 = v.astype(jnp.bfloat16)
    r1 = v - s1.astype(jnp.float32)
    s2 = r1.astype(jnp.bfloat16)
    r2 = r1 - s2.astype(jnp.float32)
    s3 = r2.astype(jnp.bfloat16)
    return s1, s2, s3


def _dist_argmin_body(x_ref, cb_ref, idx_ref, loss_ref, quant_ref,
                      cba_ref, iota_ref, cbq_ref):
    i = pl.program_id(0)
    x = x_ref[:]
    t = x.shape[0]
    d = x.shape[1]

    @pl.when(i == 0)
    def _prep():
        cb = cb_ref[:]
        c2 = jnp.sum(cb * cb, axis=1, keepdims=True)
        c1s, c2s, c3s = _split3(-2.0 * cb)
        q1, q2, q3 = _split3(c2)
        cba_ref[:] = jnp.concatenate(
            [c1s, c2s, c2s, c1s, c3s, c1s, q1, q2, q3], axis=1)
        iota_ref[:] = lax.broadcasted_iota(
            jnp.int32, iota_ref.shape, 1).astype(jnp.float32)
        e1 = cb.astype(jnp.bfloat16)
        e2 = (cb - e1.astype(jnp.float32)).astype(jnp.bfloat16)
        cbq_ref[:] = jnp.concatenate([e1, e2], axis=1)

    x1, x2, x3 = _split3(x)
    ones = jnp.ones((t, 3), jnp.bfloat16)
    xa = jnp.concatenate([x1, x2, x1, x2, x1, x3, ones], axis=1)
    score = lax.dot_general(
        xa, cba_ref[:], (((1,), (1,)), ((), ())),
        preferred_element_type=jnp.float32)
    m = jnp.min(score, axis=1, keepdims=True)
    n_k = score.shape[1]
    idx = jnp.min(jnp.where(score == m, iota_ref[:], float(n_k)),
                  axis=1, keepdims=True)
    idx_ref[:] = idx.astype(jnp.int32)

    oh = (iota_ref[:] == idx).astype(jnp.bfloat16)
    qt = lax.dot_general(
        oh, cbq_ref[:], (((1,), (0,)), ((), ())),
        preferred_element_type=jnp.float32)
    quant_ref[:] = qt[:, :d] + qt[:, d:]

    x2 = jnp.sum(x * x, axis=1, keepdims=True)
    part = jnp.sum(m + x2)

    @pl.when(i == 0)
    def _init():
        loss_ref[0] = 0.0

    loss_ref[0] = loss_ref[0] + part


def _distances_and_argmin(x, codebook):
    b, d = x.shape
    k = codebook.shape[0]
    tile = min(ROW_TILE, b)
    grid = (b // tile,)
    idx, loss_sum, quant = pl.pallas_call(
        _dist_argmin_body,
        grid=grid,
        in_specs=[
            pl.BlockSpec((tile, d), lambda i: (i, 0)),
            pl.BlockSpec((k, d), lambda i: (0, 0)),
        ],
        out_specs=[
            pl.BlockSpec((tile, 1), lambda i: (i, 0)),
            pl.BlockSpec(memory_space=pltpu.SMEM),
            pl.BlockSpec((tile, d), lambda i: (i, 0)),
        ],
        out_shape=[
            jax.ShapeDtypeStruct((b, 1), jnp.int32),
            jax.ShapeDtypeStruct((1,), jnp.float32),
            jax.ShapeDtypeStruct((b, d), jnp.float32),
        ],
        scratch_shapes=[pltpu.VMEM((k, 6 * d + 3), jnp.bfloat16),
                        pltpu.VMEM((tile, k), jnp.float32),
                        pltpu.VMEM((k, 2 * d), jnp.bfloat16)],
    )(x, codebook)
    return idx.reshape(b), loss_sum[0] / (b * d), quant


def _sc_gather(codebook, idx):
    k, d = codebook.shape
    b = idx.shape[0]
    info = plsc.get_sparse_core_info()
    nc, ns = info.num_cores, info.num_subcores
    nw = nc * ns
    b_per_w = b // nw
    ch = 128
    n_ch = b_per_w // ch
    idx2d = idx.reshape(b // ch, ch)
    mesh = plsc.VectorSubcoreMesh(core_axis_name="c", subcore_axis_name="s")

    @functools.partial(
        pl.kernel,
        mesh=mesh,
        compiler_params=pltpu.CompilerParams(use_tc_tiling_on_sc=False),
        out_type=jax.ShapeDtypeStruct((b, d), jnp.float32),
        scratch_types=[
            pltpu.VMEM((n_ch, ch), jnp.int32),
            pltpu.VMEM((b_per_w, d), jnp.float32),
            pltpu.SemaphoreType.DMA,
            pltpu.SemaphoreType.DMA,
        ],
    )
    def gather_kernel(table_hbm, idx_hbm, out_hbm, idx_v, rows_v, sem_g, sem_o):
        wid = lax.axis_index("s") * nc + lax.axis_index("c")
        base = wid * b_per_w
        pltpu.sync_copy(idx_hbm.at[pl.ds(wid * n_ch, n_ch)], idx_v)
        gathers = []
        for j in range(n_ch):
            gathers.append(pltpu.async_copy(
                table_hbm.at[idx_v.at[j]],
                rows_v.at[pl.ds(j * ch, ch)], sem_g))
        outs = []
        for j in range(n_ch):
            gathers[j].wait()
            outs.append(pltpu.async_copy(
                rows_v.at[pl.ds(j * ch, ch)],
                out_hbm.at[pl.ds(base + j * ch, ch)], sem_o))
        for c in outs:
            c.wait()

    return gather_kernel(codebook, idx2d)


def kernel(x, codebook):
    idx, loss, quant = _distances_and_argmin(x, codebook)
    return quant, loss, loss, idx
